# Initial kernel scaffold; baseline (speedup 1.0000x reference)
import jax
import jax.numpy as jnp
from jax.experimental import pallas as pl


def kernel(x, Wr, br, W1, b1, W2, b2, W3, b3, SW1, Sb1, SW2, Sb2, SW3, Sb3):
    raise NotImplementedError("write your pallas kernel here")


# speedup vs baseline: 3.0460x; 3.0460x over previous
import functools

import jax
import jax.numpy as jnp
from jax.experimental import pallas as pl
from jax.experimental.pallas import tpu as pltpu

INTERP = False

_NEG = -1e30


def _router_body(x_ref, w_ref, b_ref, i1_ref, i2_ref, g1_ref, g2_ref,
                 ps_ref, cnt_ref, z_ref):
    logits = jnp.dot(x_ref[...], w_ref[...],
                     preferred_element_type=jnp.float32) + b_ref[...]
    lane = jax.lax.broadcasted_iota(jnp.int32, logits.shape, 1)
    m1 = jnp.max(logits, axis=1, keepdims=True)
    i1 = jnp.min(jnp.where(logits == m1, lane, logits.shape[1]), axis=1,
                 keepdims=True)
    masked = jnp.where(lane == i1, _NEG, logits)
    m2 = jnp.max(masked, axis=1, keepdims=True)
    i2 = jnp.min(jnp.where(masked == m2, lane, logits.shape[1]), axis=1,
                 keepdims=True)
    d = jnp.exp(m2 - m1)
    g1_ref[...] = 1.0 / (1.0 + d)
    g2_ref[...] = d / (1.0 + d)
    i1_ref[...] = i1
    i2_ref[...] = i2
    pexp = jnp.exp(logits - m1)
    sexp = jnp.sum(pexp, axis=1, keepdims=True)
    ps_ref[...] = jnp.sum(pexp / sexp, axis=0, keepdims=True)
    lse = m1 + jnp.log(sexp)
    z_ref[...] = jnp.sum(lse * lse, axis=0, keepdims=True)
    oh = (lane == i1).astype(jnp.float32) + (lane == i2).astype(jnp.float32)
    cnt_ref[...] = jnp.sum(oh, axis=0, keepdims=True)


def _router(x_flat, Wr, br):
    T, Dd = x_flat.shape
    Ee = Wr.shape[1]
    EP = 128
    Wr_p = jnp.zeros((Dd, EP), Wr.dtype).at[:, :Ee].set(Wr)
    br_p = jnp.full((1, EP), _NEG, jnp.float32).at[0, :Ee].set(br)
    outs = (
        jax.ShapeDtypeStruct((T, 1), jnp.int32),
        jax.ShapeDtypeStruct((T, 1), jnp.int32),
        jax.ShapeDtypeStruct((T, 1), jnp.float32),
        jax.ShapeDtypeStruct((T, 1), jnp.float32),
        jax.ShapeDtypeStruct((1, EP), jnp.float32),
        jax.ShapeDtypeStruct((1, EP), jnp.float32),
        jax.ShapeDtypeStruct((1, 1), jnp.float32),
    )
    return pl.pallas_call(_router_body, out_shape=outs, interpret=INTERP)(
        x_flat, Wr_p, br_p)


def _ffn_body(gid_ref, nu_ref, x_ref, w1_ref, b1_ref, w3_ref, b3_ref,
              w2_ref, b2_ref, y_ref):
    i = pl.program_id(0)

    @pl.when(i < nu_ref[0])
    def _():
        xb = x_ref[...]
        h1 = jnp.dot(xb, w1_ref[0], preferred_element_type=jnp.float32) \
            + b1_ref[0]
        h3 = jnp.dot(xb, w3_ref[0], preferred_element_type=jnp.float32) \
            + b3_ref[0]
        h = h1 * jax.lax.logistic(h1) * h3
        y_ref[...] = jnp.dot(h, w2_ref[0], preferred_element_type=jnp.float32) \
            + b2_ref[0]


def _expert_ffn(x_pad, W1, b1, W3, b3, W2, b2, gid, nused, blk, nblk):
    Ee, Dd, Ff = W1.shape
    b1r = b1.reshape(Ee, 1, Ff)
    b3r = b3.reshape(Ee, 1, Ff)
    b2r = b2.reshape(Ee, 1, Dd)
    grid_spec = pltpu.PrefetchScalarGridSpec(
        num_scalar_prefetch=2,
        grid=(nblk,),
        in_specs=[
            pl.BlockSpec((blk, Dd), lambda i, g, n: (i, 0)),
            pl.BlockSpec((1, Dd, Ff), lambda i, g, n: (g[i], 0, 0)),
            pl.BlockSpec((1, 1, Ff), lambda i, g, n: (g[i], 0, 0)),
            pl.BlockSpec((1, Dd, Ff), lambda i, g, n: (g[i], 0, 0)),
            pl.BlockSpec((1, 1, Ff), lambda i, g, n: (g[i], 0, 0)),
            pl.BlockSpec((1, Ff, Dd), lambda i, g, n: (g[i], 0, 0)),
            pl.BlockSpec((1, 1, Dd), lambda i, g, n: (g[i], 0, 0)),
        ],
        out_specs=pl.BlockSpec((blk, Dd), lambda i, g, n: (i, 0)),
    )
    return pl.pallas_call(
        _ffn_body, grid_spec=grid_spec,
        out_shape=jax.ShapeDtypeStruct((nblk * blk, Dd), jnp.float32),
        interpret=INTERP,
    )(gid, nused, x_pad, W1, b1r, W3, b3r, W2, b2r)


def _shared_body(x_ref, w1_ref, b1_ref, w3_ref, b3_ref, w2_ref, b2_ref, o_ref):
    xb = x_ref[...]
    h1 = jnp.dot(xb, w1_ref[...], preferred_element_type=jnp.float32) \
        + b1_ref[...]
    h3 = jnp.dot(xb, w3_ref[...], preferred_element_type=jnp.float32) \
        + b3_ref[...]
    h = h1 * jax.lax.logistic(h1) * h3
    o_ref[...] = jnp.dot(h, w2_ref[...], preferred_element_type=jnp.float32) \
        + b2_ref[...]


def _shared_ffn(xin, SW1, Sb1, SW3, Sb3, SW2, Sb2, tblk=256):
    T, Dd = xin.shape
    Ff = SW1.shape[1]
    return pl.pallas_call(
        _shared_body,
        grid=(T // tblk,),
        in_specs=[
            pl.BlockSpec((tblk, Dd), lambda i: (i, 0)),
            pl.BlockSpec((Dd, Ff), lambda i: (0, 0)),
            pl.BlockSpec((1, Ff), lambda i: (0, 0)),
            pl.BlockSpec((Dd, Ff), lambda i: (0, 0)),
            pl.BlockSpec((1, Ff), lambda i: (0, 0)),
            pl.BlockSpec((Ff, Dd), lambda i: (0, 0)),
            pl.BlockSpec((1, Dd), lambda i: (0, 0)),
        ],
        out_specs=pl.BlockSpec((tblk, Dd), lambda i: (i, 0)),
        out_shape=jax.ShapeDtypeStruct((T, Dd), jnp.float32),
        interpret=INTERP,
    )(xin, SW1, Sb1.reshape(1, Ff), SW3, Sb3.reshape(1, Ff), SW2,
      Sb2.reshape(1, Dd))


def kernel(x, Wr, br, W1, b1, W2, b2, W3, b3, SW1, Sb1, SW2, Sb2, SW3, Sb3):
    Bb, Ss, Dd = x.shape
    T = Bb * Ss
    Ee = Wr.shape[1]
    Kk = 2
    BLK = 256
    NBLK = T * Kk // BLK + Ee
    x_flat = x.reshape(T, Dd)

    i1, i2, g1, g2, psum, cnt, zsum = _router(x_flat, Wr, br)
    counts_f = cnt[0, :Ee]
    z_loss = zsum[0, 0] / T
    p_mean = psum[0, :Ee] / T
    f_frac = counts_f / (T * Kk)
    lb_loss = Ee * jnp.sum(p_mean * f_frac)

    eflat = jnp.concatenate([i1, i2], axis=1).reshape(-1)
    counts_i = counts_f.astype(jnp.int32)
    starts = jnp.concatenate(
        [jnp.zeros((1,), jnp.int32), jnp.cumsum(counts_i)[:-1]])
    nblk_e = (counts_i + BLK - 1) // BLK
    blk_start = jnp.concatenate(
        [jnp.zeros((1,), jnp.int32), jnp.cumsum(nblk_e)[:-1]])
    padded_start = blk_start * BLK
    num_used = jnp.sum(nblk_e).astype(jnp.int32).reshape(1)
    order = jnp.argsort(eflat, stable=True)
    e_sorted = eflat[order]
    pos_sorted = padded_start[e_sorted] + (
        jnp.arange(T * Kk, dtype=jnp.int32) - starts[e_sorted])
    pos = jnp.zeros((T * Kk,), jnp.int32).at[order].set(pos_sorted)
    bidx = jnp.arange(NBLK, dtype=jnp.int32)
    gid = jnp.sum((blk_start[None, :] <= bidx[:, None]).astype(jnp.int32),
                  axis=1) - 1
    gid = jnp.where(bidx < num_used[0], gid,
                    gid[jnp.maximum(num_used[0] - 1, 0)])

    tok = jnp.arange(T * Kk, dtype=jnp.int32) // Kk
    x_pad = jnp.zeros((NBLK * BLK, Dd), jnp.float32).at[pos].set(x_flat[tok])

    y_pad = _expert_ffn(x_pad, W1, b1, W3, b3, W2, b2, gid, num_used,
                        BLK, NBLK)

    pos2 = pos.reshape(T, Kk)
    gates = jnp.concatenate([g1, g2], axis=1)
    comb = jnp.einsum('tk,tkd->td', gates, y_pad[pos2],
                      preferred_element_type=jnp.float32)

    out = _shared_ffn(comb, SW1, Sb1, SW3, Sb3, SW2, Sb2)
    return (out.reshape(Bb, Ss, Dd), f_frac, z_loss, z_loss * 0.001,
            lb_loss, lb_loss * 0.1)


# speedup vs baseline: 4.8641x; 1.5969x over previous
import functools

import jax
import jax.numpy as jnp
from jax import lax
from jax.experimental import pallas as pl
from jax.experimental.pallas import tpu as pltpu
from jax.experimental.pallas import tpu_sc as plsc

INTERP = False

_NEG = -1e30
_L = 16


def _dg(vec, idx):
    return lax.gather(
        vec, idx[:, None],
        lax.GatherDimensionNumbers(
            offset_dims=(), collapsed_slice_dims=(0,), start_index_map=(0,)),
        slice_sizes=(1,),
        mode=lax.GatherScatterMode.PROMISE_IN_BOUNDS)


def _router_body(x_ref, w_ref, b_ref, e_ref, g_ref, ps_ref, cnt_ref, z_ref):
    logits = jnp.dot(x_ref[...], w_ref[...],
                     preferred_element_type=jnp.float32) + b_ref[...]
    lane = lax.broadcasted_iota(jnp.int32, logits.shape, 1)
    m1 = jnp.max(logits, axis=1, keepdims=True)
    i1 = jnp.min(jnp.where(logits == m1, lane, logits.shape[1]), axis=1,
                 keepdims=True)
    masked = jnp.where(lane == i1, _NEG, logits)
    m2 = jnp.max(masked, axis=1, keepdims=True)
    i2 = jnp.min(jnp.where(masked == m2, lane, logits.shape[1]), axis=1,
                 keepdims=True)
    d = jnp.exp(m2 - m1)
    g1 = 1.0 / (1.0 + d)
    g2 = d / (1.0 + d)
    e_ref[...] = jnp.concatenate([i1, i2], axis=1)
    g_ref[...] = jnp.concatenate([g1, g2], axis=1)
    pexp = jnp.exp(logits - m1)
    sexp = jnp.sum(pexp, axis=1, keepdims=True)
    ps_ref[...] = jnp.sum(pexp / sexp, axis=0, keepdims=True)
    lse = m1 + jnp.log(sexp)
    z_ref[...] = jnp.sum(lse * lse, axis=0, keepdims=True)
    oh = (lane == i1).astype(jnp.float32) + (lane == i2).astype(jnp.float32)
    cnt_ref[...] = jnp.sum(oh, axis=0, keepdims=True)


def _router(x_flat, Wr, br):
    T, Dd = x_flat.shape
    Ee = Wr.shape[1]
    EP = 128
    Wr_p = jnp.zeros((Dd, EP), Wr.dtype).at[:, :Ee].set(Wr)
    br_p = jnp.full((1, EP), _NEG, jnp.float32).at[0, :Ee].set(br)
    outs = (
        jax.ShapeDtypeStruct((T, 2), jnp.int32),
        jax.ShapeDtypeStruct((T, 2), jnp.float32),
        jax.ShapeDtypeStruct((1, EP), jnp.float32),
        jax.ShapeDtypeStruct((1, EP), jnp.float32),
        jax.ShapeDtypeStruct((1, 1), jnp.float32),
    )
    return pl.pallas_call(_router_body, out_shape=outs, interpret=INTERP)(
        x_flat, Wr_p, br_p)


def _dispatch(eids, counts16, x_flat, blk, nblk_max):
    TK = eids.shape[0]
    T, Dd = x_flat.shape
    NW = 32
    CH = TK // NW
    NCV = CH // _L
    G = 32
    NG = CH // G
    mesh = plsc.VectorSubcoreMesh(core_axis_name="c", subcore_axis_name="s")

    @functools.partial(
        pl.kernel, mesh=mesh,
        compiler_params=pltpu.CompilerParams(needs_layout_passes=False),
        out_type=[
            jax.ShapeDtypeStruct((TK,), jnp.int32),
            jax.ShapeDtypeStruct((32,), jnp.int32),
            jax.ShapeDtypeStruct((16,), jnp.int32),
            jax.ShapeDtypeStruct((nblk_max * blk, Dd), jnp.float32),
        ],
        scratch_types=[
            pltpu.VMEM((TK,), jnp.int32),
            pltpu.VMEM((16,), jnp.int32),
            pltpu.VMEM((CH,), jnp.int32),
            pltpu.VMEM((NG, G), jnp.int32),
            pltpu.VMEM((NG, G), jnp.int32),
            pltpu.VMEM((G, Dd), jnp.float32),
            pltpu.VMEM((32,), jnp.int32),
            pltpu.VMEM((16,), jnp.int32),
            pltpu.SemaphoreType.DMA,
        ],
    )
    def body(eids_hbm, cnt_hbm, x_hbm, pos_hbm, gid_hbm, nused_hbm, xpad_hbm,
             eids_v, cnt_v, pos_v, posg_v, tokg_v, rows_v, gid_v, nu_v, sem):
        wid = lax.axis_index("s") * 2 + lax.axis_index("c")
        lane = lax.broadcasted_iota(jnp.int32, (_L,), 0)
        pltpu.sync_copy(eids_hbm, eids_v)
        pltpu.sync_copy(cnt_hbm, cnt_v)
        counts = cnt_v[...]

        def scan_body(v, before):
            ids = eids_v[pl.ds(v * _L, _L)]
            add = jnp.zeros((_L,), jnp.int32)
            for e in range(8):
                c = jnp.sum(jnp.where(ids == e, 1, 0))
                add = jnp.where(lane == e, c, add)
            return before + add

        before = lax.fori_loop(0, wid * NCV, scan_body,
                               jnp.zeros((_L,), jnp.int32))

        nblk = (counts + (blk - 1)) >> 8
        incl = plsc.cumsum(nblk)
        blk_start = incl - nblk
        padded_start = blk_start * blk
        base = padded_start + before

        my_pair = wid * CH
        for i in range(NCV):
            ids = eids_v[pl.ds(my_pair + i * _L, _L)]
            baseg = _dg(base, ids)
            rank = jnp.zeros((_L,), jnp.int32)
            add = jnp.zeros((_L,), jnp.int32)
            for e in range(8):
                m = ids == e
                mi = jnp.where(m, 1, 0)
                cs = plsc.cumsum(mi)
                rank = jnp.where(m, cs - 1, rank)
                add = jnp.where(lane == e, cs[_L - 1], add)
            pos = baseg + rank
            base = base + add
            pos_v[pl.ds(i * _L, _L)] = pos
            posg_v[i // 2, pl.ds((i % 2) * _L, _L)] = pos
            tokg_v[i // 2, pl.ds((i % 2) * _L, _L)] = \
                (my_pair + i * _L + lane) >> 1
        pltpu.sync_copy(pos_v, pos_hbm.at[pl.ds(my_pair, CH)])

        for j in range(NG):
            pltpu.async_copy(x_hbm.at[tokg_v.at[j]], rows_v, sem).wait()
            pltpu.async_copy(rows_v, xpad_hbm.at[posg_v.at[j]], sem).wait()

        @pl.when(wid == 0)
        def _():
            nused = jnp.sum(nblk)
            lastg = jnp.max(jnp.where(nblk > 0, lane, -1))
            for r in range(2):
                bv = lane + r * _L
                gv = jnp.zeros((_L,), jnp.int32)
                for e in range(8):
                    st_e = _dg(blk_start, jnp.full((_L,), e, jnp.int32))
                    gv = gv + jnp.where(st_e <= bv, 1, 0)
                gv = jnp.minimum(gv - 1, lastg)
                gid_v[pl.ds(r * _L, _L)] = gv
            nu_v[...] = jnp.where(lane == 0, nused, 0)
            pltpu.sync_copy(gid_v, gid_hbm)
            pltpu.sync_copy(nu_v, nused_hbm)

    return body(eids, counts16, x_flat)


def _combine(y_pad, pos, gates_flat, T, Dd):
    TK = pos.shape[0]
    NW = 32
    PW = TK // NW
    TW = PW // 2
    G = 32
    NG = PW // G
    TG = G // 2
    NSL = Dd // _L
    mesh = plsc.VectorSubcoreMesh(core_axis_name="c", subcore_axis_name="s")

    @functools.partial(
        pl.kernel, mesh=mesh,
        compiler_params=pltpu.CompilerParams(needs_layout_passes=False),
        out_type=jax.ShapeDtypeStruct((T, Dd), jnp.float32),
        scratch_types=[
            pltpu.VMEM((PW,), jnp.int32),
            pltpu.VMEM((PW + _L,), jnp.float32),
            pltpu.VMEM((G, Dd), jnp.float32),
            pltpu.VMEM((TG, Dd), jnp.float32),
            pltpu.SemaphoreType.DMA,
        ],
    )
    def body(y_hbm, pos_hbm, g_hbm, out_hbm, pos_v, g_v, rows_v, out_v, sem):
        wid = lax.axis_index("s") * 2 + lax.axis_index("c")
        my_pair = wid * PW
        my_tok = wid * TW
        pltpu.sync_copy(pos_hbm.at[pl.ds(my_pair, PW)], pos_v)
        pltpu.sync_copy(g_hbm.at[pl.ds(my_pair, PW)], g_v.at[pl.ds(0, PW)])
        for j in range(NG):
            pltpu.async_copy(
                y_hbm.at[pos_v.at[pl.ds(j * G, G)]], rows_v, sem).wait()

            def tok_body(t, _):
                gv = g_v[pl.ds(j * G + 2 * t, _L)]
                g0 = gv[0]
                g1 = gv[1]
                for sl in range(NSL):
                    out_v[t, pl.ds(sl * _L, _L)] = (
                        g0 * rows_v[2 * t, pl.ds(sl * _L, _L)]
                        + g1 * rows_v[2 * t + 1, pl.ds(sl * _L, _L)])
                return 0

            lax.fori_loop(0, TG, tok_body, 0)
            pltpu.sync_copy(out_v, out_hbm.at[pl.ds(my_tok + j * TG, TG)])

    return body(y_pad, pos, gates_flat)


def _ffn_body(gid_ref, nu_ref, x_ref, w1_ref, b1_ref, w3_ref, b3_ref,
              w2_ref, b2_ref, y_ref):
    i = pl.program_id(0)

    @pl.when(i < nu_ref[0])
    def _():
        xb = x_ref[...]
        h1 = jnp.dot(xb, w1_ref[0], preferred_element_type=jnp.float32) \
            + b1_ref[0]
        h3 = jnp.dot(xb, w3_ref[0], preferred_element_type=jnp.float32) \
            + b3_ref[0]
        h = h1 * lax.logistic(h1) * h3
        y_ref[...] = jnp.dot(h, w2_ref[0], preferred_element_type=jnp.float32) \
            + b2_ref[0]


def _expert_ffn(x_pad, W1, b1, W3, b3, W2, b2, gid, nused, blk, nblk):
    Ee, Dd, Ff = W1.shape
    b1r = b1.reshape(Ee, 1, Ff)
    b3r = b3.reshape(Ee, 1, Ff)
    b2r = b2.reshape(Ee, 1, Dd)
    grid_spec = pltpu.PrefetchScalarGridSpec(
        num_scalar_prefetch=2,
        grid=(nblk,),
        in_specs=[
            pl.BlockSpec((blk, Dd), lambda i, g, n: (i, 0)),
            pl.BlockSpec((1, Dd, Ff), lambda i, g, n: (g[i], 0, 0)),
            pl.BlockSpec((1, 1, Ff), lambda i, g, n: (g[i], 0, 0)),
            pl.BlockSpec((1, Dd, Ff), lambda i, g, n: (g[i], 0, 0)),
            pl.BlockSpec((1, 1, Ff), lambda i, g, n: (g[i], 0, 0)),
            pl.BlockSpec((1, Ff, Dd), lambda i, g, n: (g[i], 0, 0)),
            pl.BlockSpec((1, 1, Dd), lambda i, g, n: (g[i], 0, 0)),
        ],
        out_specs=pl.BlockSpec((blk, Dd), lambda i, g, n: (i, 0)),
    )
    return pl.pallas_call(
        _ffn_body, grid_spec=grid_spec,
        out_shape=jax.ShapeDtypeStruct((nblk * blk, Dd), jnp.float32),
        interpret=INTERP,
    )(gid, nused, x_pad, W1, b1r, W3, b3r, W2, b2r)


def _shared_body(x_ref, w1_ref, b1_ref, w3_ref, b3_ref, w2_ref, b2_ref, o_ref):
    xb = x_ref[...]
    h1 = jnp.dot(xb, w1_ref[...], preferred_element_type=jnp.float32) \
        + b1_ref[...]
    h3 = jnp.dot(xb, w3_ref[...], preferred_element_type=jnp.float32) \
        + b3_ref[...]
    h = h1 * lax.logistic(h1) * h3
    o_ref[...] = jnp.dot(h, w2_ref[...], preferred_element_type=jnp.float32) \
        + b2_ref[...]


def _shared_ffn(xin, SW1, Sb1, SW3, Sb3, SW2, Sb2, tblk=256):
    T, Dd = xin.shape
    Ff = SW1.shape[1]
    return pl.pallas_call(
        _shared_body,
        grid=(T // tblk,),
        in_specs=[
            pl.BlockSpec((tblk, Dd), lambda i: (i, 0)),
            pl.BlockSpec((Dd, Ff), lambda i: (0, 0)),
            pl.BlockSpec((1, Ff), lambda i: (0, 0)),
            pl.BlockSpec((Dd, Ff), lambda i: (0, 0)),
            pl.BlockSpec((1, Ff), lambda i: (0, 0)),
            pl.BlockSpec((Ff, Dd), lambda i: (0, 0)),
            pl.BlockSpec((1, Dd), lambda i: (0, 0)),
        ],
        out_specs=pl.BlockSpec((tblk, Dd), lambda i: (i, 0)),
        out_shape=jax.ShapeDtypeStruct((T, Dd), jnp.float32),
        interpret=INTERP,
    )(xin, SW1, Sb1.reshape(1, Ff), SW3, Sb3.reshape(1, Ff), SW2,
      Sb2.reshape(1, Dd))


def kernel(x, Wr, br, W1, b1, W2, b2, W3, b3, SW1, Sb1, SW2, Sb2, SW3, Sb3):
    Bb, Ss, Dd = x.shape
    T = Bb * Ss
    Ee = Wr.shape[1]
    Kk = 2
    BLK = 256
    NBLK = T * Kk // BLK + Ee
    x_flat = x.reshape(T, Dd)

    eids2, gates2, psum, cnt, zsum = _router(x_flat, Wr, br)
    counts_f = cnt[0, :Ee]
    z_loss = zsum[0, 0] / T
    p_mean = psum[0, :Ee] / T
    f_frac = counts_f / (T * Kk)
    lb_loss = Ee * jnp.sum(p_mean * f_frac)

    counts16 = jnp.zeros((16,), jnp.int32).at[:Ee].set(
        counts_f.astype(jnp.int32))
    eflat = eids2.reshape(T * Kk)
    gflat = gates2.reshape(T * Kk)

    pos, gid, nused, x_pad = _dispatch(eflat, counts16, x_flat, BLK, NBLK)
    y_pad = _expert_ffn(x_pad, W1, b1, W3, b3, W2, b2, gid, nused, BLK, NBLK)
    comb = _combine(y_pad, pos, gflat, T, Dd)
    out = _shared_ffn(comb, SW1, Sb1, SW3, Sb3, SW2, Sb2)
    return (out.reshape(Bb, Ss, Dd), f_frac, z_loss, z_loss * 0.001,
            lb_loss, lb_loss * 0.1)
